# baseline (speedup 1.0000x reference)
import functools

import jax
import jax.numpy as jnp
from jax import lax
from jax.experimental import pallas as pl
from jax.experimental.pallas import tpu as pltpu
from jax.experimental.pallas import tpu_sc as plsc

_B = 4096
_D = 1024
_NC = 2
_NS = 16
_NW = _NC * _NS
_L = 16
_BPW = _B // _NW
_CH = 8
_NCHUNK = _BPW // _CH
_NBUF = 5
_UNROLL = 8


@functools.partial(
    pl.kernel,
    out_type=jax.ShapeDtypeStruct((_NW, _L), jnp.float32),
    mesh=plsc.VectorSubcoreMesh(core_axis_name="c", subcore_axis_name="s"),
    scratch_types=[
        pltpu.VMEM((_BPW,), jnp.int32),
        pltpu.VMEM((_CH, _D), jnp.float32),
        pltpu.VMEM((_CH, _D), jnp.float32),
        pltpu.VMEM((_CH, _D), jnp.float32),
        pltpu.VMEM((_CH, _D), jnp.float32),
        pltpu.VMEM((_CH, _D), jnp.float32),
        pltpu.VMEM((_CH, _D), jnp.float32),
        pltpu.VMEM((_CH, _D), jnp.float32),
        pltpu.VMEM((_CH, _D), jnp.float32),
        pltpu.VMEM((_CH, _D), jnp.float32),
        pltpu.VMEM((_CH, _D), jnp.float32),
        pltpu.VMEM((_L,), jnp.float32),
        pltpu.SemaphoreType.DMA,
        pltpu.SemaphoreType.DMA,
        pltpu.SemaphoreType.DMA,
        pltpu.SemaphoreType.DMA,
        pltpu.SemaphoreType.DMA,
        pltpu.SemaphoreType.DMA,
        pltpu.SemaphoreType.DMA,
        pltpu.SemaphoreType.DMA,
        pltpu.SemaphoreType.DMA,
        pltpu.SemaphoreType.DMA,
    ],
)
def _center_loss_partials(x_hbm, y_hbm, tab_hbm, out_hbm,
                          idx_v, xb0, xb1, xb2, xb3, xb4,
                          cb0, cb1, cb2, cb3, cb4, accv,
                          sx0, sx1, sx2, sx3, sx4,
                          sc0, sc1, sc2, sc3, sc4):
    wid = lax.axis_index("s") * _NC + lax.axis_index("c")
    base = wid * _BPW

    xbs = (xb0, xb1, xb2, xb3, xb4)
    cbs = (cb0, cb1, cb2, cb3, cb4)
    sxs = (sx0, sx1, sx2, sx3, sx4)
    scs = (sc0, sc1, sc2, sc3, sc4)

    def start_x(ch, b):
        row0 = base + ch * _CH
        pltpu.async_copy(x_hbm.at[pl.ds(row0, _CH)], xbs[b], sxs[b])

    def start_c(ch, b):
        pltpu.async_copy(tab_hbm.at[idx_v.at[pl.ds(ch * _CH, _CH)]],
                         cbs[b], scs[b])

    def start(ch, b):
        start_x(ch, b)
        start_c(ch, b)

    def wait(b):
        pltpu.make_async_copy(x_hbm.at[pl.ds(0, _CH)], xbs[b],
                              sxs[b]).wait()
        pltpu.make_async_copy(tab_hbm.at[pl.ds(0, _CH)], cbs[b],
                              scs[b]).wait()

    def compute(b, acc):
        xb, cb = xbs[b], cbs[b]

        def row_body(r, a):
            def jj_body(jj, a2):
                col = jj * (_UNROLL * _L)
                p0 = None
                p1 = None
                for k in range(_UNROLL):
                    xv = xb[r, pl.ds(col + k * _L, _L)]
                    cv = cb[r, pl.ds(col + k * _L, _L)]
                    dv = xv - cv
                    dd = dv * dv
                    if k % 2 == 0:
                        p0 = dd if p0 is None else p0 + dd
                    else:
                        p1 = dd if p1 is None else p1 + dd
                return a2 + (p0 + p1)

            return lax.fori_loop(0, _D // (_UNROLL * _L), jj_body, a)

        return lax.fori_loop(0, _CH, row_body, acc)

    for b in range(_NBUF - 1):
        start_x(b, b)
    pltpu.sync_copy(y_hbm.at[pl.ds(base, _BPW)], idx_v)
    for b in range(_NBUF - 1):
        start_c(b, b)

    def penta_body(t, acc):
        ch0 = _NBUF * t
        for b in range(_NBUF):
            wait(b)
            acc = compute(b, acc)
            nxt = ch0 + b + (_NBUF - 1)

            @pl.when(nxt < _NCHUNK)
            def _():
                start(nxt, (b + _NBUF - 1) % _NBUF)

        return acc

    acc = lax.fori_loop(0, _NCHUNK // _NBUF, penta_body,
                        jnp.zeros((_L,), jnp.float32))
    for ch in range(_NCHUNK - _NCHUNK % _NBUF, _NCHUNK):
        b = ch % _NBUF
        wait(b)
        acc = compute(b, acc)

    accv[...] = acc
    pltpu.sync_copy(accv, out_hbm.at[wid])


def kernel(output_features, y_truth, feature_centers):
    batch = y_truth.shape[0]
    x = output_features.reshape(batch, -1)
    partials = _center_loss_partials(
        x, y_truth.astype(jnp.int32), feature_centers)
    return (0.5 / batch) * jnp.sum(partials)


# speedup vs baseline: 1.0339x; 1.0339x over previous
import functools

import jax
import jax.numpy as jnp
from jax import lax
from jax.experimental import pallas as pl
from jax.experimental.pallas import tpu as pltpu
from jax.experimental.pallas import tpu_sc as plsc

_B = 4096
_D = 1024
_NC = 2
_NS = 16
_NW = _NC * _NS
_L = 16
_BPW = _B // _NW
_CH = 8
_NCHUNK = _BPW // _CH
_NBUF = 4
_UNROLL = 8


@functools.partial(
    pl.kernel,
    out_type=jax.ShapeDtypeStruct((_NW, _L), jnp.float32),
    mesh=plsc.VectorSubcoreMesh(core_axis_name="c", subcore_axis_name="s"),
    scratch_types=[
        pltpu.VMEM((_BPW,), jnp.int32),
        pltpu.VMEM((_CH, _D), jnp.float32),
        pltpu.VMEM((_CH, _D), jnp.float32),
        pltpu.VMEM((_CH, _D), jnp.float32),
        pltpu.VMEM((_CH, _D), jnp.float32),
        pltpu.VMEM((_CH, _D), jnp.float32),
        pltpu.VMEM((_CH, _D), jnp.float32),
        pltpu.VMEM((_CH, _D), jnp.float32),
        pltpu.VMEM((_CH, _D), jnp.float32),
        pltpu.VMEM((_L,), jnp.float32),
        pltpu.SemaphoreType.DMA,
        pltpu.SemaphoreType.DMA,
        pltpu.SemaphoreType.DMA,
        pltpu.SemaphoreType.DMA,
        pltpu.SemaphoreType.DMA,
        pltpu.SemaphoreType.DMA,
        pltpu.SemaphoreType.DMA,
        pltpu.SemaphoreType.DMA,
    ],
)
def _center_loss_partials(x_hbm, y_hbm, tab_hbm, out_hbm,
                          idx_v, xb0, cb0, xb1, cb1, xb2, cb2, xb3, cb3,
                          accv,
                          sx0, sc0, sx1, sc1, sx2, sc2, sx3, sc3):
    wid = lax.axis_index("s") * _NC + lax.axis_index("c")
    base = wid * _BPW

    xbs = (xb0, xb1, xb2, xb3)
    cbs = (cb0, cb1, cb2, cb3)
    sxs = (sx0, sx1, sx2, sx3)
    scs = (sc0, sc1, sc2, sc3)

    def start(ch, b):
        row0 = base + ch * _CH
        pltpu.async_copy(x_hbm.at[pl.ds(row0, _CH)], xbs[b], sxs[b])
        pltpu.async_copy(tab_hbm.at[idx_v.at[pl.ds(ch * _CH, _CH)]],
                         cbs[b], scs[b])

    def wait(b):
        pltpu.make_async_copy(x_hbm.at[pl.ds(0, _CH)], xbs[b], sxs[b]).wait()
        pltpu.make_async_copy(tab_hbm.at[pl.ds(0, _CH)], cbs[b], scs[b]).wait()

    def compute(b, acc):
        xb, cb = xbs[b], cbs[b]

        def row_body(r, a):
            def jj_body(jj, a2):
                col = jj * (_UNROLL * _L)
                p0 = None
                p1 = None
                for k in range(_UNROLL):
                    xv = xb[r, pl.ds(col + k * _L, _L)]
                    cv = cb[r, pl.ds(col + k * _L, _L)]
                    dv = xv - cv
                    dd = dv * dv
                    if k % 2 == 0:
                        p0 = dd if p0 is None else p0 + dd
                    else:
                        p1 = dd if p1 is None else p1 + dd
                return a2 + (p0 + p1)

            return lax.fori_loop(0, _D // (_UNROLL * _L), jj_body, a)

        return lax.fori_loop(0, _CH, row_body, acc)

    for b in range(_NBUF - 1):
        row0 = base + b * _CH
        pltpu.async_copy(x_hbm.at[pl.ds(row0, _CH)], xbs[b], sxs[b])
    pltpu.sync_copy(y_hbm.at[pl.ds(base, _BPW)], idx_v)
    for b in range(_NBUF - 1):
        pltpu.async_copy(tab_hbm.at[idx_v.at[pl.ds(b * _CH, _CH)]],
                         cbs[b], scs[b])

    def quad_body(t, acc):
        ch0 = 4 * t
        for b in range(_NBUF):
            wait(b)
            acc = compute(b, acc)
            nxt = ch0 + b + (_NBUF - 1)

            @pl.when(nxt < _NCHUNK)
            def _():
                start(nxt, (b + 3) % _NBUF)

        return acc

    acc = lax.fori_loop(0, _NCHUNK // _NBUF, quad_body,
                        jnp.zeros((_L,), jnp.float32))
    accv[...] = acc
    pltpu.sync_copy(accv, out_hbm.at[wid])


def kernel(output_features, y_truth, feature_centers):
    batch = y_truth.shape[0]
    x = output_features.reshape(batch, -1)
    partials = _center_loss_partials(
        x, y_truth.astype(jnp.int32), feature_centers)
    return (0.5 / batch) * jnp.sum(partials)
